# Initial kernel scaffold; baseline (speedup 1.0000x reference)
import functools

import jax
import jax.numpy as jnp
from jax import lax
from jax.experimental import pallas as pl
from jax.experimental.pallas import tpu as pltpu
from jax.experimental.pallas import tpu_sc as plsc

WIN = 128
NSUB = 16
RB = 4096


def _ceil_to(x, m):
    return (x + m - 1) // m * m


def _sc_mesh():
    return plsc.VectorSubcoreMesh(core_axis_name="c", subcore_axis_name="s")


def _sc_degree(dstp, zeros, npad, epw):
    nwin = epw // WIN
    rows_per_sub = npad // NSUB

    @functools.partial(
        pl.kernel,
        out_type=jax.ShapeDtypeStruct((npad, 16), jnp.float32),
        mesh=_sc_mesh(),
        scratch_types=[
            pltpu.VMEM((WIN,), jnp.int32),
            pltpu.VMEM((WIN, 16), jnp.float32),
            pltpu.VMEM_SHARED((npad, 16), jnp.float32),
            pltpu.SemaphoreType.DMA,
        ],
    )
    def k(dst_hbm, zeros_hbm, p_hbm, didx, ones, acc, sem):
        c = lax.axis_index("c")
        s = lax.axis_index("s")
        rbase = pl.multiple_of(s * rows_per_sub, 8)
        ebase = pl.multiple_of(s * epw, 8)

        @pl.when(c == 0)
        def _():
            @pl.loop(0, WIN)
            def _(i):
                ones[i, :] = jnp.full((16,), 1.0, jnp.float32)

            pltpu.sync_copy(zeros_hbm.at[pl.ds(rbase, rows_per_sub)],
                            acc.at[pl.ds(rbase, rows_per_sub)])
            plsc.subcore_barrier()

            @pl.loop(0, nwin)
            def _(w):
                base = pl.multiple_of(ebase + w * WIN, 8)
                pltpu.sync_copy(dst_hbm.at[pl.ds(base, WIN)], didx)
                pltpu.sync_copy(ones, acc.at[didx], add=True)

            plsc.subcore_barrier()
            pltpu.sync_copy(acc.at[pl.ds(rbase, rows_per_sub)],
                            p_hbm.at[pl.ds(rbase, rows_per_sub)])

    return k(dstp, zeros)


def _sc_aggregate(gl, gr, srcp, dstp, npad, epw):
    nwin = epw // WIN
    rows_per_sub = npad // NSUB

    @functools.partial(
        pl.kernel,
        out_type=[jax.ShapeDtypeStruct((npad, 32), jnp.float32)] * 2,
        mesh=_sc_mesh(),
        scratch_types=[
            pltpu.VMEM((WIN,), jnp.int32),
            pltpu.VMEM((WIN,), jnp.int32),
            pltpu.VMEM((WIN, 32), jnp.float32),
            pltpu.VMEM_SHARED((npad, 32), jnp.float32),
            pltpu.SemaphoreType.DMA,
        ],
    )
    def k(gl_hbm, gr_hbm, src_hbm, dst_hbm, ol_hbm, or_hbm,
          sidx, didx, rows, acc, sem):
        c = lax.axis_index("c")
        s = lax.axis_index("s")
        rbase = pl.multiple_of(s * rows_per_sub, 8)
        ebase = pl.multiple_of(s * epw, 8)

        def run(g_hbm, o_hbm):
            pltpu.sync_copy(g_hbm.at[pl.ds(rbase, rows_per_sub)],
                            acc.at[pl.ds(rbase, rows_per_sub)])
            plsc.subcore_barrier()

            @pl.loop(0, nwin)
            def _(w):
                base = pl.multiple_of(ebase + w * WIN, 8)
                pltpu.sync_copy(src_hbm.at[pl.ds(base, WIN)], sidx)
                pltpu.sync_copy(dst_hbm.at[pl.ds(base, WIN)], didx)
                pltpu.async_copy(g_hbm.at[sidx], rows, sem).wait()
                pltpu.sync_copy(rows, acc.at[didx], add=True)

            plsc.subcore_barrier()
            pltpu.sync_copy(acc.at[pl.ds(rbase, rows_per_sub)],
                            o_hbm.at[pl.ds(rbase, rows_per_sub)])

        @pl.when(c == 0)
        def _():
            run(gl_hbm, ol_hbm)

        @pl.when(c == 1)
        def _():
            run(gr_hbm, or_hbm)

    return k(gl, gr, srcp, dstp)


_PREC = lax.Precision.HIGHEST


def _tc_prep(xp, p0, npad):
    grid = npad // RB

    def body(x_ref, p_ref, gl_ref, gr_ref, dinv_ref):
        dinv = lax.rsqrt(p_ref[:, 0:1] + 1.0)
        g = x_ref[...] * dinv
        gl_ref[...] = g[:, :32]
        gr_ref[...] = g[:, 32:]
        dinv_ref[...] = jnp.broadcast_to(dinv, (RB, 8))

    return pl.pallas_call(
        body,
        grid=(grid,),
        in_specs=[
            pl.BlockSpec((RB, 64), lambda i: (i, 0)),
            pl.BlockSpec((RB, 16), lambda i: (i, 0)),
        ],
        out_specs=[
            pl.BlockSpec((RB, 32), lambda i: (i, 0)),
            pl.BlockSpec((RB, 32), lambda i: (i, 0)),
            pl.BlockSpec((RB, 8), lambda i: (i, 0)),
        ],
        out_shape=[
            jax.ShapeDtypeStruct((npad, 32), jnp.float32),
            jax.ShapeDtypeStruct((npad, 32), jnp.float32),
            jax.ShapeDtypeStruct((npad, 8), jnp.float32),
        ],
    )(xp, p0)


def _tc_mid(al, ar, dinv, w, b, npad, relu):
    grid = npad // RB

    def body(al_ref, ar_ref, dinv_ref, w_ref, b_ref, gl_ref, gr_ref):
        d = dinv_ref[:, 0:1]
        h = jnp.concatenate([al_ref[...], ar_ref[...]], axis=1) * d
        h = lax.dot_general(h, w_ref[...], (((1,), (1,)), ((), ())),
                            preferred_element_type=jnp.float32,
                            precision=_PREC) + b_ref[...]
        if relu:
            h = jnp.maximum(h, 0.0)
        g = h * d
        gl_ref[...] = g[:, :32]
        gr_ref[...] = g[:, 32:]

    return pl.pallas_call(
        body,
        grid=(grid,),
        in_specs=[
            pl.BlockSpec((RB, 32), lambda i: (i, 0)),
            pl.BlockSpec((RB, 32), lambda i: (i, 0)),
            pl.BlockSpec((RB, 8), lambda i: (i, 0)),
            pl.BlockSpec((64, 64), lambda i: (0, 0)),
            pl.BlockSpec((1, 64), lambda i: (0, 0)),
        ],
        out_specs=[
            pl.BlockSpec((RB, 32), lambda i: (i, 0)),
            pl.BlockSpec((RB, 32), lambda i: (i, 0)),
        ],
        out_shape=[
            jax.ShapeDtypeStruct((npad, 32), jnp.float32),
            jax.ShapeDtypeStruct((npad, 32), jnp.float32),
        ],
    )(al, ar, dinv, w, b)


def _tc_final(al, ar, dinv, w2, b2, n, out_dim):
    grid = (n + RB - 1) // RB

    def body(al_ref, ar_ref, dinv_ref, w_ref, b_ref, o_ref):
        d = dinv_ref[:, 0:1]
        h = jnp.concatenate([al_ref[...], ar_ref[...]], axis=1) * d
        o = lax.dot_general(h, w_ref[...], (((1,), (1,)), ((), ())),
                            preferred_element_type=jnp.float32,
                            precision=_PREC) + b_ref[...]
        m = jnp.max(o, axis=1, keepdims=True)
        e = o - m
        lse = jnp.log(jnp.sum(jnp.exp(e), axis=1, keepdims=True))
        o_ref[...] = e - lse

    return pl.pallas_call(
        body,
        grid=(grid,),
        in_specs=[
            pl.BlockSpec((RB, 32), lambda i: (i, 0)),
            pl.BlockSpec((RB, 32), lambda i: (i, 0)),
            pl.BlockSpec((RB, 8), lambda i: (i, 0)),
            pl.BlockSpec((out_dim, 64), lambda i: (0, 0)),
            pl.BlockSpec((1, out_dim), lambda i: (0, 0)),
        ],
        out_specs=pl.BlockSpec((RB, out_dim), lambda i: (i, 0)),
        out_shape=jax.ShapeDtypeStruct((n, out_dim), jnp.float32),
    )(al, ar, dinv, w2, b2)


def kernel(x, edge_index, W1, b1, W2, b2):
    n, in_dim = x.shape
    e = edge_index.shape[1]
    hid = W1.shape[0]
    out_dim = W2.shape[0]
    assert in_dim == 64 and hid == 64

    npad = _ceil_to(n + 1, max(RB, 128))
    epw = _ceil_to(-(-e // NSUB), WIN)
    epad = epw * NSUB

    src = edge_index[0].astype(jnp.int32)
    dst = edge_index[1].astype(jnp.int32)
    pad_idx = jnp.full((epad - e,), n, dtype=jnp.int32)
    srcp = jnp.concatenate([src, pad_idx])
    dstp = jnp.concatenate([dst, pad_idx])

    xp = jnp.zeros((npad, in_dim), jnp.float32).at[:n].set(x)
    zeros16 = jnp.zeros((npad, 16), jnp.float32)
    b1r = b1.reshape(1, hid)
    b2r = b2.reshape(1, out_dim)

    p0 = _sc_degree(dstp, zeros16, npad, epw)
    gl, gr, dinv = _tc_prep(xp, p0, npad)

    a1l, a1r = _sc_aggregate(gl, gr, srcp, dstp, npad, epw)
    g2l, g2r = _tc_mid(a1l, a1r, dinv, W1, b1r, npad, relu=False)

    a2l, a2r = _sc_aggregate(g2l, g2r, srcp, dstp, npad, epw)
    g3l, g3r = _tc_mid(a2l, a2r, dinv, W1, b1r, npad, relu=True)

    a3l, a3r = _sc_aggregate(g3l, g3r, srcp, dstp, npad, epw)
    return _tc_final(a3l, a3r, dinv, W2, b2r, n, out_dim)


# speedup vs baseline: 10.0585x; 10.0585x over previous
import functools

import jax
import jax.numpy as jnp
from jax import lax
from jax.experimental import pallas as pl
from jax.experimental.pallas import tpu as pltpu
from jax.experimental.pallas import tpu_sc as plsc

WIN = 128
NSUB = 16
RB = 4096


def _ceil_to(x, m):
    return (x + m - 1) // m * m


def _sc_mesh():
    return plsc.VectorSubcoreMesh(core_axis_name="c", subcore_axis_name="s")


_SC_PARAMS = pltpu.CompilerParams(use_tc_tiling_on_sc=False)


def _sc_degree(dstp, zeros, npad, epw):
    nwin = epw // WIN
    rows_per_sub = npad // NSUB

    @functools.partial(
        pl.kernel,
        out_type=jax.ShapeDtypeStruct((npad, 16), jnp.float32),
        mesh=_sc_mesh(),
        scratch_types=[
            pltpu.VMEM((WIN,), jnp.int32),
            pltpu.VMEM((WIN, 16), jnp.float32),
            pltpu.VMEM_SHARED((npad, 16), jnp.float32),
            pltpu.SemaphoreType.DMA,
        ],
        compiler_params=_SC_PARAMS,
    )
    def k(dst_hbm, zeros_hbm, p_hbm, didx, ones, acc, sem):
        c = lax.axis_index("c")
        s = lax.axis_index("s")
        rbase = pl.multiple_of(s * rows_per_sub, 8)
        ebase = pl.multiple_of(s * epw, 8)

        @pl.when(c == 0)
        def _():
            @pl.loop(0, WIN)
            def _(i):
                ones[i, :] = jnp.full((16,), 1.0, jnp.float32)

            pltpu.sync_copy(zeros_hbm.at[pl.ds(rbase, rows_per_sub)],
                            acc.at[pl.ds(rbase, rows_per_sub)])
            plsc.subcore_barrier()

            @pl.loop(0, nwin)
            def _(w):
                base = pl.multiple_of(ebase + w * WIN, 8)
                pltpu.sync_copy(dst_hbm.at[pl.ds(base, WIN)], didx)
                pltpu.sync_copy(ones, acc.at[didx], add=True)

            plsc.subcore_barrier()
            pltpu.sync_copy(acc.at[pl.ds(rbase, rows_per_sub)],
                            p_hbm.at[pl.ds(rbase, rows_per_sub)])

    return k(dstp, zeros)


def _sc_aggregate(gl, gr, srcp, dstp, npad, epw):
    nwin = epw // WIN
    rows_per_sub = npad // NSUB

    @functools.partial(
        pl.kernel,
        out_type=[jax.ShapeDtypeStruct((npad, 32), jnp.float32)] * 2,
        mesh=_sc_mesh(),
        scratch_types=[
            pltpu.VMEM((WIN,), jnp.int32),
            pltpu.VMEM((WIN,), jnp.int32),
            pltpu.VMEM((WIN, 32), jnp.float32),
            pltpu.VMEM_SHARED((npad, 32), jnp.float32),
            pltpu.SemaphoreType.DMA,
        ],
        compiler_params=_SC_PARAMS,
    )
    def k(gl_hbm, gr_hbm, src_hbm, dst_hbm, ol_hbm, or_hbm,
          sidx, didx, rows, acc, sem):
        c = lax.axis_index("c")
        s = lax.axis_index("s")
        rbase = pl.multiple_of(s * rows_per_sub, 8)
        ebase = pl.multiple_of(s * epw, 8)

        def run(g_hbm, o_hbm):
            pltpu.sync_copy(g_hbm.at[pl.ds(rbase, rows_per_sub)],
                            acc.at[pl.ds(rbase, rows_per_sub)])
            plsc.subcore_barrier()

            @pl.loop(0, nwin)
            def _(w):
                base = pl.multiple_of(ebase + w * WIN, 8)
                pltpu.sync_copy(src_hbm.at[pl.ds(base, WIN)], sidx)
                pltpu.sync_copy(dst_hbm.at[pl.ds(base, WIN)], didx)
                pltpu.async_copy(g_hbm.at[sidx], rows, sem).wait()
                pltpu.sync_copy(rows, acc.at[didx], add=True)

            plsc.subcore_barrier()
            pltpu.sync_copy(acc.at[pl.ds(rbase, rows_per_sub)],
                            o_hbm.at[pl.ds(rbase, rows_per_sub)])

        @pl.when(c == 0)
        def _():
            run(gl_hbm, ol_hbm)

        @pl.when(c == 1)
        def _():
            run(gr_hbm, or_hbm)

    return k(gl, gr, srcp, dstp)


_PREC = lax.Precision.HIGHEST


def _tc_prep(xp, p0, npad):
    grid = npad // RB

    def body(x_ref, p_ref, gl_ref, gr_ref, dinv_ref):
        dinv = lax.rsqrt(p_ref[:, 0:1] + 1.0)
        g = x_ref[...] * dinv
        gl_ref[...] = g[:, :32]
        gr_ref[...] = g[:, 32:]
        dinv_ref[...] = jnp.broadcast_to(dinv, (RB, 8))

    return pl.pallas_call(
        body,
        grid=(grid,),
        in_specs=[
            pl.BlockSpec((RB, 64), lambda i: (i, 0)),
            pl.BlockSpec((RB, 16), lambda i: (i, 0)),
        ],
        out_specs=[
            pl.BlockSpec((RB, 32), lambda i: (i, 0)),
            pl.BlockSpec((RB, 32), lambda i: (i, 0)),
            pl.BlockSpec((RB, 8), lambda i: (i, 0)),
        ],
        out_shape=[
            jax.ShapeDtypeStruct((npad, 32), jnp.float32),
            jax.ShapeDtypeStruct((npad, 32), jnp.float32),
            jax.ShapeDtypeStruct((npad, 8), jnp.float32),
        ],
    )(xp, p0)


def _tc_mid(al, ar, dinv, w, b, npad, relu):
    grid = npad // RB

    def body(al_ref, ar_ref, dinv_ref, w_ref, b_ref, gl_ref, gr_ref):
        d = dinv_ref[:, 0:1]
        h = jnp.concatenate([al_ref[...], ar_ref[...]], axis=1) * d
        h = lax.dot_general(h, w_ref[...], (((1,), (1,)), ((), ())),
                            preferred_element_type=jnp.float32,
                            precision=_PREC) + b_ref[...]
        if relu:
            h = jnp.maximum(h, 0.0)
        g = h * d
        gl_ref[...] = g[:, :32]
        gr_ref[...] = g[:, 32:]

    return pl.pallas_call(
        body,
        grid=(grid,),
        in_specs=[
            pl.BlockSpec((RB, 32), lambda i: (i, 0)),
            pl.BlockSpec((RB, 32), lambda i: (i, 0)),
            pl.BlockSpec((RB, 8), lambda i: (i, 0)),
            pl.BlockSpec((64, 64), lambda i: (0, 0)),
            pl.BlockSpec((1, 64), lambda i: (0, 0)),
        ],
        out_specs=[
            pl.BlockSpec((RB, 32), lambda i: (i, 0)),
            pl.BlockSpec((RB, 32), lambda i: (i, 0)),
        ],
        out_shape=[
            jax.ShapeDtypeStruct((npad, 32), jnp.float32),
            jax.ShapeDtypeStruct((npad, 32), jnp.float32),
        ],
    )(al, ar, dinv, w, b)


def _tc_final(al, ar, dinv, w2, b2, n, out_dim):
    grid = (n + RB - 1) // RB

    def body(al_ref, ar_ref, dinv_ref, w_ref, b_ref, o_ref):
        d = dinv_ref[:, 0:1]
        h = jnp.concatenate([al_ref[...], ar_ref[...]], axis=1) * d
        o = lax.dot_general(h, w_ref[...], (((1,), (1,)), ((), ())),
                            preferred_element_type=jnp.float32,
                            precision=_PREC) + b_ref[...]
        m = jnp.max(o, axis=1, keepdims=True)
        e = o - m
        lse = jnp.log(jnp.sum(jnp.exp(e), axis=1, keepdims=True))
        o_ref[...] = e - lse

    return pl.pallas_call(
        body,
        grid=(grid,),
        in_specs=[
            pl.BlockSpec((RB, 32), lambda i: (i, 0)),
            pl.BlockSpec((RB, 32), lambda i: (i, 0)),
            pl.BlockSpec((RB, 8), lambda i: (i, 0)),
            pl.BlockSpec((out_dim, 64), lambda i: (0, 0)),
            pl.BlockSpec((1, out_dim), lambda i: (0, 0)),
        ],
        out_specs=pl.BlockSpec((RB, out_dim), lambda i: (i, 0)),
        out_shape=jax.ShapeDtypeStruct((n, out_dim), jnp.float32),
    )(al, ar, dinv, w2, b2)


def kernel(x, edge_index, W1, b1, W2, b2):
    n, in_dim = x.shape
    e = edge_index.shape[1]
    hid = W1.shape[0]
    out_dim = W2.shape[0]
    assert in_dim == 64 and hid == 64

    npad = _ceil_to(n + 1, max(RB, 128))
    epw = _ceil_to(-(-e // NSUB), WIN)
    epad = epw * NSUB

    src = edge_index[0].astype(jnp.int32)
    dst = edge_index[1].astype(jnp.int32)
    pad_idx = jnp.full((epad - e,), n, dtype=jnp.int32)
    srcp = jnp.concatenate([src, pad_idx])
    dstp = jnp.concatenate([dst, pad_idx])

    xp = jnp.zeros((npad, in_dim), jnp.float32).at[:n].set(x)
    zeros16 = jnp.zeros((npad, 16), jnp.float32)
    b1r = b1.reshape(1, hid)
    b2r = b2.reshape(1, out_dim)

    p0 = _sc_degree(dstp, zeros16, npad, epw)
    gl, gr, dinv = _tc_prep(xp, p0, npad)

    a1l, a1r = _sc_aggregate(gl, gr, srcp, dstp, npad, epw)
    g2l, g2r = _tc_mid(a1l, a1r, dinv, W1, b1r, npad, relu=False)

    a2l, a2r = _sc_aggregate(g2l, g2r, srcp, dstp, npad, epw)
    g3l, g3r = _tc_mid(a2l, a2r, dinv, W1, b1r, npad, relu=True)

    a3l, a3r = _sc_aggregate(g3l, g3r, srcp, dstp, npad, epw)
    return _tc_final(a3l, a3r, dinv, W2, b2r, n, out_dim)


# speedup vs baseline: 19.1477x; 1.9036x over previous
import functools

import jax
import jax.numpy as jnp
from jax import lax
from jax.experimental import pallas as pl
from jax.experimental.pallas import tpu as pltpu
from jax.experimental.pallas import tpu_sc as plsc

WIN = 128
KAGG = 2
NSUB = 16
RB = 4096


def _ceil_to(x, m):
    return (x + m - 1) // m * m


def _sc_mesh():
    return plsc.VectorSubcoreMesh(core_axis_name="c", subcore_axis_name="s")


_SC_PARAMS = pltpu.CompilerParams(use_tc_tiling_on_sc=False)


def _sc_aggregate(gl, gr, srcp2, dstp2, npad, epw):
    rows_per_sub = npad // NSUB
    wrows = epw // WIN
    pairs = wrows // (2 * KAGG)
    SW = KAGG * WIN

    @functools.partial(
        pl.kernel,
        out_type=[jax.ShapeDtypeStruct((npad, 32), jnp.float32)] * 2,
        mesh=_sc_mesh(),
        scratch_types=[
            pltpu.VMEM((KAGG, WIN), jnp.int32),
            pltpu.VMEM((KAGG, WIN), jnp.int32),
            pltpu.VMEM((KAGG, WIN), jnp.int32),
            pltpu.VMEM((KAGG, WIN), jnp.int32),
            pltpu.VMEM((SW, 32), jnp.float32),
            pltpu.VMEM((SW, 32), jnp.float32),
            pltpu.VMEM_SHARED((npad, 32), jnp.float32),
            pltpu.SemaphoreType.DMA,
            pltpu.SemaphoreType.DMA,
            pltpu.SemaphoreType.DMA,
            pltpu.SemaphoreType.DMA,
            pltpu.SemaphoreType.DMA,
            pltpu.SemaphoreType.DMA,
        ],
        compiler_params=_SC_PARAMS,
    )
    def k(gl_hbm, gr_hbm, src_hbm, dst_hbm, ol_hbm, or_hbm,
          sidxa, didxa, sidxb, didxb, rowsa, rowsb, acc,
          semia, semib, semga, semgb, semsa, semsb):
        c = lax.axis_index("c")
        s = lax.axis_index("s")
        rbase = pl.multiple_of(s * rows_per_sub, 8)
        wbase = pl.multiple_of(s * wrows, 8)

        def idx_fetch(swg, sidx, didx, semi):
            rb = wbase + swg * KAGG
            pltpu.async_copy(src_hbm.at[pl.ds(rb, KAGG)], sidx, semi)
            pltpu.async_copy(dst_hbm.at[pl.ds(rb, KAGG)], didx, semi)

        def idx_wait(sidx, didx, semi):
            pltpu.make_async_copy(src_hbm.at[pl.ds(0, KAGG)], sidx, semi).wait()
            pltpu.make_async_copy(dst_hbm.at[pl.ds(0, KAGG)], didx, semi).wait()

        def run(g_hbm, o_hbm):
            pltpu.sync_copy(g_hbm.at[pl.ds(rbase, rows_per_sub)],
                            acc.at[pl.ds(rbase, rows_per_sub)])
            plsc.subcore_barrier()

            def gather_fire(sidx, rows, semg):
                return [pltpu.async_copy(g_hbm.at[sidx.at[kk]],
                                         rows.at[pl.ds(kk * WIN, WIN)], semg)
                        for kk in range(KAGG)]

            def scatter_fire(didx, rows, sems):
                for kk in range(KAGG):
                    pltpu.async_copy(rows.at[pl.ds(kk * WIN, WIN)],
                                     acc.at[didx.at[kk]], sems, add=True)

            def scatter_drain(didx, rows, sems):
                for kk in range(KAGG):
                    pltpu.make_async_copy(rows.at[pl.ds(kk * WIN, WIN)],
                                          acc.at[didx.at[kk]], sems).wait()

            idx_fetch(0, sidxa, didxa, semia)
            idx_fetch(1, sidxb, didxb, semib)

            @pl.loop(0, pairs)
            def _(i):
                @pl.when(i > 0)
                def _():
                    scatter_drain(didxb, rowsb, semsb)
                    idx_fetch(2 * i + 1, sidxb, didxb, semib)

                idx_wait(sidxa, didxa, semia)
                for h in gather_fire(sidxa, rowsa, semga):
                    h.wait()
                scatter_fire(didxa, rowsa, semsa)

                idx_wait(sidxb, didxb, semib)
                hb = gather_fire(sidxb, rowsb, semgb)
                scatter_drain(didxa, rowsa, semsa)

                @pl.when(i < pairs - 1)
                def _():
                    idx_fetch(2 * i + 2, sidxa, didxa, semia)

                for h in hb:
                    h.wait()
                scatter_fire(didxb, rowsb, semsb)

            scatter_drain(didxb, rowsb, semsb)
            plsc.subcore_barrier()
            pltpu.sync_copy(acc.at[pl.ds(rbase, rows_per_sub)],
                            o_hbm.at[pl.ds(rbase, rows_per_sub)])

        @pl.when(c == 0)
        def _():
            run(gl_hbm, ol_hbm)

        @pl.when(c == 1)
        def _():
            run(gr_hbm, or_hbm)

    return k(gl, gr, srcp2, dstp2)


_PREC = lax.Precision.HIGHEST


def _tc_prep(xp, d0, npad):
    grid = npad // RB

    def body(x_ref, p0_ref, gl_ref, gr_ref, dinv_ref):
        dinv = lax.rsqrt(p0_ref[:, 0:1])
        g = x_ref[...] * dinv
        gl_ref[...] = g[:, :32]
        gr_ref[...] = g[:, 32:]
        dinv_ref[...] = jnp.broadcast_to(dinv, (RB, 8))

    return pl.pallas_call(
        body,
        grid=(grid,),
        in_specs=[
            pl.BlockSpec((RB, 64), lambda i: (i, 0)),
            pl.BlockSpec((RB, 32), lambda i: (i, 0)),
        ],
        out_specs=[
            pl.BlockSpec((RB, 32), lambda i: (i, 0)),
            pl.BlockSpec((RB, 32), lambda i: (i, 0)),
            pl.BlockSpec((RB, 8), lambda i: (i, 0)),
        ],
        out_shape=[
            jax.ShapeDtypeStruct((npad, 32), jnp.float32),
            jax.ShapeDtypeStruct((npad, 32), jnp.float32),
            jax.ShapeDtypeStruct((npad, 8), jnp.float32),
        ],
    )(xp, d0)


def _tc_mid(al, ar, dinv, w, b, npad, relu):
    grid = npad // RB

    def body(al_ref, ar_ref, dinv_ref, w_ref, b_ref, gl_ref, gr_ref):
        d = dinv_ref[:, 0:1]
        h = jnp.concatenate([al_ref[...], ar_ref[...]], axis=1) * d
        h = lax.dot_general(h, w_ref[...], (((1,), (1,)), ((), ())),
                            preferred_element_type=jnp.float32,
                            precision=_PREC) + b_ref[...]
        if relu:
            h = jnp.maximum(h, 0.0)
        g = h * d
        gl_ref[...] = g[:, :32]
        gr_ref[...] = g[:, 32:]

    return pl.pallas_call(
        body,
        grid=(grid,),
        in_specs=[
            pl.BlockSpec((RB, 32), lambda i: (i, 0)),
            pl.BlockSpec((RB, 32), lambda i: (i, 0)),
            pl.BlockSpec((RB, 8), lambda i: (i, 0)),
            pl.BlockSpec((64, 64), lambda i: (0, 0)),
            pl.BlockSpec((1, 64), lambda i: (0, 0)),
        ],
        out_specs=[
            pl.BlockSpec((RB, 32), lambda i: (i, 0)),
            pl.BlockSpec((RB, 32), lambda i: (i, 0)),
        ],
        out_shape=[
            jax.ShapeDtypeStruct((npad, 32), jnp.float32),
            jax.ShapeDtypeStruct((npad, 32), jnp.float32),
        ],
    )(al, ar, dinv, w, b)


def _tc_final(al, ar, dinv, w2, b2, n, out_dim):
    grid = (n + RB - 1) // RB

    def body(al_ref, ar_ref, dinv_ref, w_ref, b_ref, o_ref):
        d = dinv_ref[:, 0:1]
        h = jnp.concatenate([al_ref[...], ar_ref[...]], axis=1) * d
        o = lax.dot_general(h, w_ref[...], (((1,), (1,)), ((), ())),
                            preferred_element_type=jnp.float32,
                            precision=_PREC) + b_ref[...]
        m = jnp.max(o, axis=1, keepdims=True)
        e = o - m
        lse = jnp.log(jnp.sum(jnp.exp(e), axis=1, keepdims=True))
        o_ref[...] = e - lse

    return pl.pallas_call(
        body,
        grid=(grid,),
        in_specs=[
            pl.BlockSpec((RB, 32), lambda i: (i, 0)),
            pl.BlockSpec((RB, 32), lambda i: (i, 0)),
            pl.BlockSpec((RB, 8), lambda i: (i, 0)),
            pl.BlockSpec((out_dim, 64), lambda i: (0, 0)),
            pl.BlockSpec((1, out_dim), lambda i: (0, 0)),
        ],
        out_specs=pl.BlockSpec((RB, out_dim), lambda i: (i, 0)),
        out_shape=jax.ShapeDtypeStruct((n, out_dim), jnp.float32),
    )(al, ar, dinv, w2, b2)


def kernel(x, edge_index, W1, b1, W2, b2):
    n, in_dim = x.shape
    e = edge_index.shape[1]
    hid = W1.shape[0]
    out_dim = W2.shape[0]
    assert in_dim == 64 and hid == 64

    npad = _ceil_to(n + 1, max(RB, 128))
    per_sub = _ceil_to(-(-e // NSUB), 2 * KAGG * WIN)
    epad = per_sub * NSUB

    src = edge_index[0].astype(jnp.int32)
    dst = edge_index[1].astype(jnp.int32)
    pad_idx = jnp.full((epad - e,), n, dtype=jnp.int32)
    srcp2 = jnp.concatenate([src, pad_idx]).reshape(-1, 128)
    dstp2 = jnp.concatenate([dst, pad_idx]).reshape(-1, 128)

    xp = jnp.zeros((npad, in_dim), jnp.float32).at[:n].set(x)
    b1r = b1.reshape(1, hid)
    b2r = b2.reshape(1, out_dim)

    ones32 = jnp.ones((npad, 32), jnp.float32)
    d0, _ = _sc_aggregate(ones32, ones32, srcp2, dstp2, npad, per_sub)
    gl, gr, dinv = _tc_prep(xp, d0, npad)

    a1l, a1r = _sc_aggregate(gl, gr, srcp2, dstp2, npad, per_sub)
    g2l, g2r = _tc_mid(a1l, a1r, dinv, W1, b1r, npad, relu=False)

    a2l, a2r = _sc_aggregate(g2l, g2r, srcp2, dstp2, npad, per_sub)
    g3l, g3r = _tc_mid(a2l, a2r, dinv, W1, b1r, npad, relu=True)

    a3l, a3r = _sc_aggregate(g3l, g3r, srcp2, dstp2, npad, per_sub)
    return _tc_final(a3l, a3r, dinv, W2, b2r, n, out_dim)
